# Initial kernel scaffold; baseline (speedup 1.0000x reference)
import jax
import jax.numpy as jnp
from jax.experimental import pallas as pl


def kernel(coords, segment_ids):
    raise NotImplementedError("write your pallas kernel here")


# speedup vs baseline: 29.4675x; 29.4675x over previous
import functools

import jax
import jax.numpy as jnp
from jax import lax
from jax.experimental import pallas as pl
from jax.experimental.pallas import tpu as pltpu
from jax.experimental.pallas import tpu_sc as plsc

_M = 131072
_NW = 32
_S = _M // _NW
_CH = 2048
_CH_SHIFT = 11
_GROW = 128
_NROW = _S // _GROW

_params = pltpu.CompilerParams(
    needs_layout_passes=False, use_tc_tiling_on_sc=False)


def _wid():
  return lax.axis_index("s") * 2 + lax.axis_index("c")


def _mesh():
  return plsc.VectorSubcoreMesh(core_axis_name="c", subcore_axis_name="s")


@functools.cache
def _make_sums_kernel():
  @functools.partial(
      pl.kernel,
      mesh=_mesh(),
      out_type=jax.ShapeDtypeStruct((4 * _M,), jnp.float32),
      scratch_types=[
          pltpu.VMEM((_CH,), jnp.int32),
          pltpu.VMEM((_CH * 3,), jnp.float32),
          pltpu.VMEM((_S,), jnp.float32),
          pltpu.VMEM((_S,), jnp.float32),
          pltpu.VMEM((_S,), jnp.float32),
          pltpu.VMEM((_S,), jnp.float32),
          pltpu.VMEM((64,), jnp.int32),
      ],
      compiler_params=_params,
  )
  def sums_kernel(cflat_hbm, seg_hbm, bounds_hbm, out_s,
                  seg_v, xyz_v, sx_v, sy_v, sz_v, cnt_v, bnd_s):
    wid = _wid()
    seg_base = wid * _S
    lane = lax.broadcasted_iota(jnp.int32, (16,), 0)
    lane3 = lane * 3
    zf = jnp.zeros((16,), jnp.float32)
    onesf = zf + 1.0

    pltpu.sync_copy(bounds_hbm, bnd_s)
    bvec = bnd_s[pl.ds(wid, 16)]
    p_lo = bvec[0]
    p_hi = bvec[1]
    c_lo = lax.shift_right_logical(p_lo, _CH_SHIFT)
    c_hi = lax.shift_right_logical(p_hi + (_CH - 1), _CH_SHIFT)

    def init_body(i, _):
      sl = pl.ds(i * 16, 16)
      sx_v[sl] = zf
      sy_v[sl] = zf
      sz_v[sl] = zf
      cnt_v[sl] = zf
      return 0

    lax.fori_loop(0, _S // 16, init_body, 0, unroll=False)

    def p1_chunk(ci, _):
      pbase = ci * _CH
      pltpu.sync_copy(seg_hbm.at[pl.ds(pbase, _CH)], seg_v)
      pltpu.sync_copy(cflat_hbm.at[pl.ds(pbase * 3, _CH * 3)], xyz_v)

      def grp(g, _):
        off = g * 16
        seg = seg_v[pl.ds(off, 16)]
        lseg = seg - seg_base
        valid = (lseg >= 0) & (lseg < _S)
        lsegc = jnp.clip(lseg, 0, _S - 1)
        base3 = lane3 + off * 3
        x = plsc.load_gather(xyz_v, [base3])
        y = plsc.load_gather(xyz_v, [base3 + 1])
        z = plsc.load_gather(xyz_v, [base3 + 2])
        plsc.addupdate_scatter(sx_v, [lsegc], x, mask=valid)
        plsc.addupdate_scatter(sy_v, [lsegc], y, mask=valid)
        plsc.addupdate_scatter(sz_v, [lsegc], z, mask=valid)
        plsc.addupdate_scatter(cnt_v, [lsegc], onesf, mask=valid)
        return 0

      lax.fori_loop(0, _CH // 16, grp, 0, unroll=False)
      return 0

    lax.fori_loop(c_lo, c_hi, p1_chunk, 0, unroll=False)

    pltpu.sync_copy(sx_v, out_s.at[pl.ds(seg_base, _S)])
    pltpu.sync_copy(sy_v, out_s.at[pl.ds(_M + seg_base, _S)])
    pltpu.sync_copy(sz_v, out_s.at[pl.ds(2 * _M + seg_base, _S)])
    pltpu.sync_copy(cnt_v, out_s.at[pl.ds(3 * _M + seg_base, _S)])

  return sums_kernel


_CAP = 4096
_CAPC = 16384


@functools.cache
def _make_expand_kernel(n_points):
  nchunks_per_tile = n_points // _CH // _NW

  @functools.partial(
      pl.kernel,
      mesh=_mesh(),
      out_type=jax.ShapeDtypeStruct((n_points * 3,), jnp.float32),
      scratch_types=[
          pltpu.VMEM((_CH + 16,), jnp.int32),
          pltpu.VMEM((_CH * 3,), jnp.float32),
          pltpu.VMEM((_CAP,), jnp.float32),
          pltpu.VMEM((_CAP,), jnp.float32),
          pltpu.VMEM((_CAP,), jnp.float32),
      ],
      compiler_params=_params,
  )
  def expand_kernel(seg_hbm, ax_hbm, ay_hbm, az_hbm, out_exp,
                    seg_v, row_v, axb, ayb, azb):
    wid = _wid()
    lane = lax.broadcasted_iota(jnp.int32, (16,), 0)
    lane3 = lane * 3

    def chunk(ci, _):
      gci = wid * nchunks_per_tile + ci
      pbase = gci * _CH
      pltpu.sync_copy(seg_hbm.at[pl.ds(pbase, _CH)], seg_v.at[pl.ds(0, _CH)])

      def cond(pstart):
        return pstart < _CH

      def body(pstart):
        svec = seg_v[pl.ds(pstart, 16)]
        c0 = jnp.minimum(svec[0] & jnp.int32(-8), jnp.int32(_M - _CAP))
        c0 = pl.multiple_of(c0, 8)
        pltpu.sync_copy(ax_hbm.at[pl.ds(c0, _CAP)], axb)
        pltpu.sync_copy(ay_hbm.at[pl.ds(c0, _CAP)], ayb)
        pltpu.sync_copy(az_hbm.at[pl.ds(c0, _CAP)], azb)

        def grp(g, nxt):
          off = g * 16
          seg = seg_v[pl.ds(off, 16)]
          lseg = seg - c0
          inpass = (lseg >= 0) & (lseg < _CAP)
          lsegc = jnp.clip(lseg, 0, _CAP - 1)
          ax = plsc.load_gather(axb, [lsegc])
          ay = plsc.load_gather(ayb, [lsegc])
          az = plsc.load_gather(azb, [lsegc])
          base3 = lane3 + off * 3
          plsc.store_scatter(row_v, [base3], ax, mask=inpass)
          plsc.store_scatter(row_v, [base3 + 1], ay, mask=inpass)
          plsc.store_scatter(row_v, [base3 + 2], az, mask=inpass)
          over = jnp.where(lseg >= _CAP, lane + off, _CH)
          return jnp.minimum(nxt, jnp.min(over))

        return lax.fori_loop(0, _CH // 16, grp, jnp.int32(_CH),
                             unroll=False)

      lax.while_loop(cond, body, jnp.int32(0))
      pltpu.sync_copy(row_v, out_exp.at[pl.ds(pbase * 3, _CH * 3)])
      return 0

    lax.fori_loop(0, nchunks_per_tile, chunk, 0, unroll=False)

  return expand_kernel


@functools.cache
def _make_argmin_kernel(n_points):
  @functools.partial(
      pl.kernel,
      mesh=_mesh(),
      out_type=(
          jax.ShapeDtypeStruct((_M * 3,), jnp.float32),
          jax.ShapeDtypeStruct((_M,), jnp.int32),
      ),
      scratch_types=[
          pltpu.VMEM((_CH,), jnp.int32),
          pltpu.VMEM((_CH,), jnp.float32),
          pltpu.VMEM((_S,), jnp.float32),
          pltpu.VMEM((_S,), jnp.int32),
          pltpu.VMEM((16,), jnp.float32),
          pltpu.VMEM((16,), jnp.int32),
          pltpu.VMEM((16,), jnp.int32),
          pltpu.VMEM((_S * 3,), jnp.float32),
          pltpu.VMEM((_CAPC * 3,), jnp.float32),
          pltpu.VMEM((64,), jnp.int32),
      ],
      compiler_params=_params,
  )
  def argmin_kernel(cflat_hbm, seg_hbm, dist_hbm, bounds_hbm,
                    out_coords, out_map,
                    seg_v, dist_v, bd2_v, bidx_v, d2_b, ix_b, seg_b,
                    rows_v, cwin_v, bnd_s):
    wid = _wid()
    seg_base = wid * _S
    lane = lax.broadcasted_iota(jnp.int32, (16,), 0)
    shifts = [jnp.maximum(lane - d, 0) for d in (1, 2, 4, 8)]
    zf = jnp.zeros((16,), jnp.float32)
    inff = zf + jnp.float32(jnp.inf)
    minint = (lane * 0) + jnp.int32(-2147483648)

    pltpu.sync_copy(bounds_hbm, bnd_s)
    bvec = bnd_s[pl.ds(wid, 16)]
    p_lo = bvec[0]
    p_hi = bvec[1]
    c_lo = lax.shift_right_logical(p_lo, _CH_SHIFT)
    c_hi = lax.shift_right_logical(p_hi + (_CH - 1), _CH_SHIFT)

    def init_body(i, _):
      sl = pl.ds(i * 16, 16)
      bd2_v[sl] = inff
      bidx_v[sl] = minint
      return 0

    lax.fori_loop(0, _S // 16, init_body, 0, unroll=False)

    def p3_chunk(ci, _):
      pbase = ci * _CH
      pltpu.sync_copy(seg_hbm.at[pl.ds(pbase, _CH)], seg_v)
      pltpu.sync_copy(dist_hbm.at[pl.ds(pbase, _CH)], dist_v)

      def grp(g, _):
        off = g * 16
        seg = seg_v[pl.ds(off, 16)]
        lseg = seg - seg_base
        valid = (lseg >= 0) & (lseg < _S)
        lsegc = jnp.clip(lseg, 0, _S - 1)
        d2 = dist_v[pl.ds(off, 16)]
        pidx = pbase + off + lane
        _, last = plsc.scan_count(seg)
        seg_b[...] = seg
        for sh in shifts:
          d2_b[...] = d2
          ix_b[...] = pidx
          seg_sh = plsc.load_gather(seg_b, [sh])
          d2_sh = plsc.load_gather(d2_b, [sh])
          ix_sh = plsc.load_gather(ix_b, [sh])
          better = (seg_sh == seg) & (d2_sh < d2)
          d2 = jnp.where(better, d2_sh, d2)
          pidx = jnp.where(better, ix_sh, pidx)
        m = last & valid
        od2 = plsc.load_gather(bd2_v, [lsegc])
        oix = plsc.load_gather(bidx_v, [lsegc])
        takec = d2 <= od2
        nd2 = jnp.where(takec, d2, od2)
        nix = jnp.where(takec, pidx, oix)
        plsc.store_scatter(bd2_v, [lsegc], nd2, mask=m)
        plsc.store_scatter(bidx_v, [lsegc], nix, mask=m)
        return 0

      lax.fori_loop(0, _CH // 16, grp, 0, unroll=False)
      return 0

    lax.fori_loop(c_lo, c_hi, p3_chunk, 0, unroll=False)

    def wcond(wbase):
      return wbase < p_hi

    def wbody(wbase):
      w_lo = jnp.minimum(wbase, jnp.int32(n_points - _CAPC))
      w_lo = pl.multiple_of(w_lo, 8)
      pltpu.sync_copy(cflat_hbm.at[pl.ds(w_lo * 3, _CAPC * 3)], cwin_v)

      def cgrp(i, _):
        base = i * 16
        idx = bidx_v[pl.ds(base, 16)]
        inwin = (idx >= w_lo) & (idx < w_lo + _CAPC)
        li3 = (idx - w_lo) * 3
        gx = plsc.load_gather(cwin_v, [li3], mask=inwin)
        gy = plsc.load_gather(cwin_v, [li3 + 1], mask=inwin)
        gz = plsc.load_gather(cwin_v, [li3 + 2], mask=inwin)
        cell3 = (base + lane) * 3
        plsc.store_scatter(rows_v, [cell3], gx, mask=inwin)
        plsc.store_scatter(rows_v, [cell3 + 1], gy, mask=inwin)
        plsc.store_scatter(rows_v, [cell3 + 2], gz, mask=inwin)
        return 0

      lax.fori_loop(0, _S // 16, cgrp, 0, unroll=False)
      return wbase + _CAPC

    w0 = pl.multiple_of(p_lo & jnp.int32(-8), 8)
    lax.while_loop(wcond, wbody, w0)

    pltpu.sync_copy(rows_v, out_coords.at[pl.ds(seg_base * 3, _S * 3)])
    pltpu.sync_copy(bidx_v, out_map.at[pl.ds(seg_base, _S)])

  return argmin_kernel


@jax.jit
def _run(coords, seg):
  n = coords.shape[0]
  seg = seg.astype(jnp.int32)
  cflat = coords.reshape(-1)
  cuts = jnp.arange(0, _M + 1, _S, dtype=jnp.int32)
  bounds = jnp.searchsorted(seg, cuts, side="left").astype(jnp.int32)
  bounds = jnp.pad(bounds, (0, 64 - bounds.shape[0]))

  sums4 = _make_sums_kernel()(cflat, seg, bounds)
  cnt = jnp.maximum(sums4[3 * _M:], 1.0)
  ax = sums4[:_M] / cnt
  ay = sums4[_M:2 * _M] / cnt
  az = sums4[2 * _M:3 * _M] / cnt

  avg_exp = _make_expand_kernel(n)(seg, ax, ay, az).reshape(n, 3)
  diff = avg_exp - coords
  dist = jnp.sqrt(jnp.sum(diff ** 2, axis=1))

  new_flat, dmap = _make_argmin_kernel(n)(cflat, seg, dist, bounds)
  new_coords = new_flat.reshape(_M, 3)
  empty = dmap == jnp.int32(-2147483648)
  new_coords = jnp.where(empty[:, None], coords[0], new_coords)
  return new_coords, dmap


def kernel(coords, segment_ids):
  return _run(coords, segment_ids)
